# baseline (speedup 1.0000x reference)
import jax
import jax.numpy as jnp
from jax import lax
from jax.experimental import pallas as pl
from jax.experimental.pallas import tpu as pltpu
from jax.experimental.pallas import tpu_sc as plsc

_NC = 2
_NS = 16
_NW = _NC * _NS
_D = 64
_E = 16
_SLAB = 512
_NBUF = 2
_BL = 1600


def _sc_gather_body(table_hbm, idx_hbm, e_hbm, table_v, idx_v, sbuf,
                    ssem0, ssem1):
    ssem = (ssem0, ssem1)
    wid = lax.axis_index("s") * _NC + lax.axis_index("c")
    rows_w = idx_hbm.shape[0] // _NW
    ngroups = rows_w // _SLAB
    erow0 = wid * (rows_w * _E // 128)
    erows_slab = _SLAB * _E // 128

    pltpu.sync_copy(table_hbm, table_v)
    pltpu.sync_copy(idx_hbm.at[pl.ds(wid * rows_w, rows_w)], idx_v)

    iota = lax.broadcasted_iota(jnp.int32, (16,), 0)
    colvecs = [(iota & 7) * _E + c for c in range(_E)]
    rowhalf = (iota >> 3)

    def scatter_copy(g, b):
        return pltpu.make_async_copy(
            sbuf.at[b],
            e_hbm.at[pl.ds(erow0 + g * erows_slab, erows_slab)],
            ssem[b],
        )

    def fill_slab(g, b):
        def trow(t, carry):
            idxv = idx_v[pl.ds(g * _SLAB + t * 16, 16)]
            srcbase = idxv * _E
            rowvec = rowhalf + t * 2
            for c in range(_E):
                vals = plsc.load_gather(table_v, [srcbase + c])
                plsc.store_scatter(sbuf.at[b], [rowvec, colvecs[c]], vals)
            return carry

        lax.fori_loop(0, _SLAB // 16, trow, 0)

    def step(i, carry):
        for b in range(_NBUF):
            g = i * _NBUF + b

            @pl.when(g >= _NBUF)
            def _():
                scatter_copy(g - _NBUF, b).wait()

            fill_slab(g, b)
            scatter_copy(g, b).start()
        return carry

    lax.fori_loop(0, ngroups // _NBUF, step, 0)
    scatter_copy(ngroups - 2, (ngroups - 2) % _NBUF).wait()
    scatter_copy(ngroups - 1, (ngroups - 1) % _NBUF).wait()


def _tc_proj_body(e_ref, w_ref, b_ref, o_ref):
    e = e_ref[...]
    for j in range(128 // _E):
        o_ref[:, j, :] = (
            jnp.dot(e[:, _E * j:_E * (j + 1)], w_ref[...],
                    preferred_element_type=jnp.float32)
            + b_ref[...]
        )


def kernel(src, table, W, b):
    B, S = src.shape
    total = B * S
    rpb = 128 // _E
    ne = total // rpb

    table16 = jnp.pad(table, ((0, 0), (0, _E - table.shape[1]))).reshape(-1)
    idx = src.reshape(total)

    e2 = pl.kernel(
        _sc_gather_body,
        out_type=jax.ShapeDtypeStruct((ne, 128), jnp.float32),
        mesh=plsc.VectorSubcoreMesh(core_axis_name="c", subcore_axis_name="s"),
        compiler_params=pltpu.CompilerParams(
            use_tc_tiling_on_sc=False, needs_layout_passes=False),
        scratch_types=[
            pltpu.VMEM(table16.shape, jnp.float32),
            pltpu.VMEM((total // _NW,), jnp.int32),
            pltpu.VMEM((_NBUF, _SLAB * _E // 128, 128), jnp.float32),
            pltpu.SemaphoreType.DMA,
            pltpu.SemaphoreType.DMA,
        ],
    )(table16, idx)

    w16 = jnp.pad(W.T, ((0, _E - W.shape[1]), (0, 0)))
    out = pl.pallas_call(
        _tc_proj_body,
        grid=(ne // _BL,),
        in_specs=[
            pl.BlockSpec((_BL, 128), lambda i: (i, 0)),
            pl.BlockSpec((_E, _D), lambda i: (0, 0)),
            pl.BlockSpec((1, _D), lambda i: (0, 0)),
        ],
        out_specs=pl.BlockSpec((_BL, rpb, _D), lambda i: (i, 0, 0)),
        out_shape=jax.ShapeDtypeStruct((ne, rpb, _D), jnp.float32),
    )(e2, w16, b[None, :])
    return out.reshape(B, S, _D)


# speedup vs baseline: 1.3280x; 1.3280x over previous
import jax
import jax.numpy as jnp
from jax import lax
from jax.experimental import pallas as pl
from jax.experimental.pallas import tpu as pltpu
from jax.experimental.pallas import tpu_sc as plsc

_NC = 2
_NS = 16
_NW = _NC * _NS
_D = 64
_E = 16
_SE = 8
_SLAB = 512
_NBUF = 2
_BL = 1600


def _sc_gather_body(table_hbm, idx_hbm, e_hbm, table_v, idx_v, sbuf,
                    ssem0, ssem1):
    ssem = (ssem0, ssem1)
    wid = lax.axis_index("s") * _NC + lax.axis_index("c")
    rows_w = idx_hbm.shape[0] // _NW
    ngroups = rows_w // _SLAB
    erow0 = wid * (rows_w * _E // 128)
    erows_slab = _SLAB * _E // 128

    pltpu.sync_copy(table_hbm, table_v)
    pltpu.sync_copy(idx_hbm.at[pl.ds(wid * rows_w, rows_w)], idx_v)

    zv = jnp.zeros((16,), jnp.float32)

    def zrow(r, carry):
        for bb in range(_NBUF):
            for l in range(128 // 16):
                sbuf[bb, r, pl.ds(l * 16, 16)] = zv
        return carry

    lax.fori_loop(0, erows_slab, zrow, 0)

    iota = lax.broadcasted_iota(jnp.int32, (16,), 0)
    colvecs = [(iota & 7) * _E + c for c in range(_SE)]
    rowhalf = iota >> 3

    def scatter_copy(g, b):
        return pltpu.make_async_copy(
            sbuf.at[b],
            e_hbm.at[pl.ds(erow0 + g * erows_slab, erows_slab)],
            ssem[b],
        )

    def fill_slab(g, b):
        @plsc.parallel_loop(0, _SLAB // 16, unroll=2)
        def trow(t):
            idxv = idx_v[pl.ds(g * _SLAB + t * 16, 16)]
            srcbase = idxv * _E
            rowvec = rowhalf + t * 2
            for c in range(_SE):
                vals = plsc.load_gather(table_v, [srcbase + c])
                plsc.store_scatter(sbuf.at[b], [rowvec, colvecs[c]], vals)

    def step(i, carry):
        for b in range(_NBUF):
            g = i * _NBUF + b

            @pl.when(g >= _NBUF)
            def _():
                scatter_copy(g - _NBUF, b).wait()

            fill_slab(g, b)
            scatter_copy(g, b).start()
        return carry

    lax.fori_loop(0, ngroups // _NBUF, step, 0)
    scatter_copy(ngroups - 2, (ngroups - 2) % _NBUF).wait()
    scatter_copy(ngroups - 1, (ngroups - 1) % _NBUF).wait()


def _tc_proj_body(e_ref, w_ref, b_ref, o_ref):
    e = e_ref[...]
    o4 = o_ref.reshape(_BL, 128 // _E, _D)
    for j in range(128 // _E):
        o4[:, j, :] = (
            jnp.dot(e[:, _E * j:_E * (j + 1)], w_ref[...],
                    preferred_element_type=jnp.float32)
            + b_ref[...]
        )


def kernel(src, table, W, b):
    B, S = src.shape
    total = B * S
    rpb = 128 // _E
    ne = total // rpb

    table16 = jnp.pad(table, ((0, 0), (0, _E - table.shape[1]))).reshape(-1)
    idx = src.reshape(total)

    e2 = pl.kernel(
        _sc_gather_body,
        out_type=jax.ShapeDtypeStruct((ne, 128), jnp.float32),
        mesh=plsc.VectorSubcoreMesh(core_axis_name="c", subcore_axis_name="s"),
        compiler_params=pltpu.CompilerParams(
            use_tc_tiling_on_sc=False, needs_layout_passes=False),
        scratch_types=[
            pltpu.VMEM(table16.shape, jnp.float32),
            pltpu.VMEM((total // _NW,), jnp.int32),
            pltpu.VMEM((_NBUF, _SLAB * _E // 128, 128), jnp.float32),
            pltpu.SemaphoreType.DMA,
            pltpu.SemaphoreType.DMA,
        ],
    )(table16, idx)

    w16 = jnp.pad(W.T, ((0, _E - W.shape[1]), (0, 0)))
    out = pl.pallas_call(
        _tc_proj_body,
        grid=(ne // _BL,),
        in_specs=[
            pl.BlockSpec((_BL, 128), lambda i: (i, 0)),
            pl.BlockSpec((_E, _D), lambda i: (0, 0)),
            pl.BlockSpec((1, _D), lambda i: (0, 0)),
        ],
        out_specs=pl.BlockSpec((_BL * rpb, _D), lambda i: (i, 0)),
        out_shape=jax.ShapeDtypeStruct((total, _D), jnp.float32),
    )(e2, w16, b[None, :])
    return out.reshape(B, S, _D)


# speedup vs baseline: 1.3372x; 1.0069x over previous
import jax
import jax.numpy as jnp
from jax import lax
from jax.experimental import pallas as pl
from jax.experimental.pallas import tpu as pltpu
from jax.experimental.pallas import tpu_sc as plsc

_NC = 2
_NS = 16
_NW = _NC * _NS
_D = 64
_E = 16
_SE = 8
_SLAB = 512
_NBUF = 2
_BL = 3200


def _sc_gather_body(table_hbm, idx_hbm, e_hbm, table_v, idx_v, sbuf,
                    ssem0, ssem1):
    ssem = (ssem0, ssem1)
    wid = lax.axis_index("s") * _NC + lax.axis_index("c")
    rows_w = idx_hbm.shape[0] // _NW
    ngroups = rows_w // _SLAB
    erow0 = wid * (rows_w * _E // 128)
    erows_slab = _SLAB * _E // 128

    pltpu.sync_copy(table_hbm, table_v)
    pltpu.sync_copy(idx_hbm.at[pl.ds(wid * rows_w, rows_w)], idx_v)

    zv = jnp.zeros((16,), jnp.float32)

    def zrow(r, carry):
        for bb in range(_NBUF):
            for l in range(128 // 16):
                sbuf[bb, r, pl.ds(l * 16, 16)] = zv
        return carry

    lax.fori_loop(0, erows_slab, zrow, 0)

    iota = lax.broadcasted_iota(jnp.int32, (16,), 0)
    colvecs = [(iota & 7) * _E + c for c in range(_SE)]
    rowhalf = iota >> 3

    def scatter_copy(g, b):
        return pltpu.make_async_copy(
            sbuf.at[b],
            e_hbm.at[pl.ds(erow0 + g * erows_slab, erows_slab)],
            ssem[b],
        )

    def fill_slab(g, b):
        @plsc.parallel_loop(0, _SLAB // 16, unroll=2)
        def trow(t):
            idxv = idx_v[pl.ds(g * _SLAB + t * 16, 16)]
            srcbase = idxv * _E
            rowvec = rowhalf + t * 2
            for c in range(_SE):
                vals = plsc.load_gather(table_v, [srcbase + c])
                plsc.store_scatter(sbuf.at[b], [rowvec, colvecs[c]], vals)

    def step(i, carry):
        for b in range(_NBUF):
            g = i * _NBUF + b

            @pl.when(g >= _NBUF)
            def _():
                scatter_copy(g - _NBUF, b).wait()

            fill_slab(g, b)
            scatter_copy(g, b).start()
        return carry

    lax.fori_loop(0, ngroups // _NBUF, step, 0)
    scatter_copy(ngroups - 2, (ngroups - 2) % _NBUF).wait()
    scatter_copy(ngroups - 1, (ngroups - 1) % _NBUF).wait()


def _tc_proj_body(e_ref, w_ref, b_ref, o_ref):
    e = e_ref[...]
    o4 = o_ref.reshape(_BL, 128 // _E, _D)
    for j in range(128 // _E):
        o4[:, j, :] = (
            jnp.dot(e[:, _E * j:_E * (j + 1)], w_ref[...],
                    preferred_element_type=jnp.float32)
            + b_ref[...]
        )


def kernel(src, table, W, b):
    B, S = src.shape
    total = B * S
    rpb = 128 // _E
    ne = total // rpb

    table16 = jnp.pad(table, ((0, 0), (0, _E - table.shape[1]))).reshape(-1)
    idx = src.reshape(total)

    e2 = pl.kernel(
        _sc_gather_body,
        out_type=jax.ShapeDtypeStruct((ne, 128), jnp.float32),
        mesh=plsc.VectorSubcoreMesh(core_axis_name="c", subcore_axis_name="s"),
        compiler_params=pltpu.CompilerParams(
            use_tc_tiling_on_sc=False, needs_layout_passes=False),
        scratch_types=[
            pltpu.VMEM(table16.shape, jnp.float32),
            pltpu.VMEM((total // _NW,), jnp.int32),
            pltpu.VMEM((_NBUF, _SLAB * _E // 128, 128), jnp.float32),
            pltpu.SemaphoreType.DMA,
            pltpu.SemaphoreType.DMA,
        ],
    )(table16, idx)

    w16 = jnp.pad(W.T, ((0, _E - W.shape[1]), (0, 0)))
    out = pl.pallas_call(
        _tc_proj_body,
        grid=(ne // _BL,),
        in_specs=[
            pl.BlockSpec((_BL, 128), lambda i: (i, 0)),
            pl.BlockSpec((_E, _D), lambda i: (0, 0)),
            pl.BlockSpec((1, _D), lambda i: (0, 0)),
        ],
        out_specs=pl.BlockSpec((_BL * rpb, _D), lambda i: (i, 0)),
        out_shape=jax.ShapeDtypeStruct((total, _D), jnp.float32),
    )(e2, w16, b[None, :])
    return out.reshape(B, S, _D)


# speedup vs baseline: 1.3390x; 1.0013x over previous
import jax
import jax.numpy as jnp
from jax import lax
from jax.experimental import pallas as pl
from jax.experimental.pallas import tpu as pltpu
from jax.experimental.pallas import tpu_sc as plsc

_NC = 2
_NS = 16
_NW = _NC * _NS
_D = 64
_E = 16
_SE = 8
_SLAB = 800
_NBUF = 2
_BL = 3200


def _sc_gather_body(table_hbm, idx_hbm, e_hbm, table_v, idx_v, sbuf,
                    ssem0, ssem1):
    ssem = (ssem0, ssem1)
    wid = lax.axis_index("s") * _NC + lax.axis_index("c")
    rows_w = idx_hbm.shape[0] // _NW
    ngroups = rows_w // _SLAB
    erow0 = wid * (rows_w * _E // 128)
    erows_slab = _SLAB * _E // 128

    pltpu.sync_copy(table_hbm, table_v)
    pltpu.sync_copy(idx_hbm.at[pl.ds(wid * rows_w, rows_w)], idx_v)

    zv = jnp.zeros((16,), jnp.float32)

    def zrow(r, carry):
        for bb in range(_NBUF):
            for l in range(128 // 16):
                sbuf[bb, r, pl.ds(l * 16, 16)] = zv
        return carry

    lax.fori_loop(0, erows_slab, zrow, 0)

    iota = lax.broadcasted_iota(jnp.int32, (16,), 0)
    colvecs = [(iota & 7) * _E + c for c in range(_SE)]
    rowhalf = iota >> 3

    def scatter_copy(g, b):
        return pltpu.make_async_copy(
            sbuf.at[b],
            e_hbm.at[pl.ds(erow0 + g * erows_slab, erows_slab)],
            ssem[b],
        )

    def fill_slab(g, b):
        @plsc.parallel_loop(0, _SLAB // 16, unroll=2)
        def trow(t):
            idxv = idx_v[pl.ds(g * _SLAB + t * 16, 16)]
            srcbase = idxv * _E
            rowvec = rowhalf + t * 2
            for c in range(_SE):
                vals = plsc.load_gather(table_v, [srcbase + c])
                plsc.store_scatter(sbuf.at[b], [rowvec, colvecs[c]], vals)

    def step(i, carry):
        for b in range(_NBUF):
            g = i * _NBUF + b

            @pl.when(g >= _NBUF)
            def _():
                scatter_copy(g - _NBUF, b).wait()

            fill_slab(g, b)
            scatter_copy(g, b).start()
        return carry

    lax.fori_loop(0, ngroups // _NBUF, step, 0)
    scatter_copy(ngroups - 2, (ngroups - 2) % _NBUF).wait()
    scatter_copy(ngroups - 1, (ngroups - 1) % _NBUF).wait()


def _tc_proj_body(e_ref, w_ref, b_ref, o_ref):
    e = e_ref[...]
    o4 = o_ref.reshape(_BL, 128 // _E, _D)
    for j in range(128 // _E):
        o4[:, j, :] = (
            jnp.dot(e[:, _E * j:_E * (j + 1)], w_ref[...],
                    preferred_element_type=jnp.float32)
            + b_ref[...]
        )


def kernel(src, table, W, b):
    B, S = src.shape
    total = B * S
    rpb = 128 // _E
    ne = total // rpb

    table16 = jnp.pad(table, ((0, 0), (0, _E - table.shape[1]))).reshape(-1)
    idx = src.reshape(total)

    e2 = pl.kernel(
        _sc_gather_body,
        out_type=jax.ShapeDtypeStruct((ne, 128), jnp.float32),
        mesh=plsc.VectorSubcoreMesh(core_axis_name="c", subcore_axis_name="s"),
        compiler_params=pltpu.CompilerParams(
            use_tc_tiling_on_sc=False, needs_layout_passes=False),
        scratch_types=[
            pltpu.VMEM(table16.shape, jnp.float32),
            pltpu.VMEM((total // _NW,), jnp.int32),
            pltpu.VMEM((_NBUF, _SLAB * _E // 128, 128), jnp.float32),
            pltpu.SemaphoreType.DMA,
            pltpu.SemaphoreType.DMA,
        ],
    )(table16, idx)

    w16 = jnp.pad(W.T, ((0, _E - W.shape[1]), (0, 0)))
    out = pl.pallas_call(
        _tc_proj_body,
        grid=(ne // _BL,),
        in_specs=[
            pl.BlockSpec((_BL, 128), lambda i: (i, 0)),
            pl.BlockSpec((_E, _D), lambda i: (0, 0)),
            pl.BlockSpec((1, _D), lambda i: (0, 0)),
        ],
        out_specs=pl.BlockSpec((_BL * rpb, _D), lambda i: (i, 0)),
        out_shape=jax.ShapeDtypeStruct((total, _D), jnp.float32),
    )(e2, w16, b[None, :])
    return out.reshape(B, S, _D)
